# Initial kernel scaffold; baseline (speedup 1.0000x reference)
import jax
import jax.numpy as jnp
from jax.experimental import pallas as pl


def kernel(x, edge_index, edge_index1, edge_index2, central_mask, W_r, W_s, b_s, W_t, b_t, Wg_s2t, Wg_t2s, Wf_s2t, Wf_t2s):
    raise NotImplementedError("write your pallas kernel here")


# speedup vs baseline: 1.1177x; 1.1177x over previous
import functools

import jax
import jax.numpy as jnp
from jax import lax
from jax.experimental import pallas as pl
from jax.experimental.pallas import tpu as pltpu
from jax.experimental.pallas import tpu_sc as plsc

N = 10000
D = 256
E = 80000
EP = 81920
NW = 32
EPW = EP // NW
KA = 64
SCN = 512
SR = 320
PAD_DST = 16000


def _stats_body(x_ref, cm_ref, mc_ref, mn_ref):
    x = x_ref[...]
    cm = cm_ref[...]
    sc = jnp.sum(cm)
    mc_ref[...] = (jnp.sum(x * cm, axis=0) / jnp.maximum(sc, 1.0))[None, :]
    mn_ref[...] = (jnp.sum(x * (1.0 - cm), axis=0)
                   / jnp.maximum(float(N) - sc, 1.0))[None, :]


def _dense_body(x_ref, cm_ref, mc_ref, mn_ref, wr_ref, ws_ref, bs_ref,
                wt_ref, bt_ref, wg1_ref, wg2_ref, u1_ref, u2_ref, xwr_ref):
    x = x_ref[...]
    cm = cm_ref[...]
    delta = mc_ref[0, :] - mn_ref[0, :]
    wg1 = wg1_ref[...]
    wg2 = wg2_ref[...]
    c1 = jnp.sum(delta * wg1[D:, 0])
    c2 = jnp.sum(delta * wg2[D:, 0])
    g1 = jnp.dot(x, wg1[:D])
    g2 = jnp.dot(x, wg2[:D])
    sh1 = jnp.tanh(g1 + c1) * delta[None, :]
    sh2 = jnp.tanh(g2 + c2) * delta[None, :]
    xs = x - sh1 * cm
    xt = x + sh2 * (1.0 - cm)
    u1_ref[...] = jnp.dot(xt, ws_ref[...]) + bs_ref[...]
    u2_ref[...] = jnp.dot(xs, wt_ref[...]) + bt_ref[...]
    xwr_ref[...] = jnp.dot(x, wr_ref[...])


def _dense_prep(x, cmf, W_r, W_s, b_s, W_t, b_t, Wg_s2t, Wg_t2s):
    mc, mn = pl.pallas_call(
        _stats_body,
        out_shape=[jax.ShapeDtypeStruct((1, D), jnp.float32)] * 2,
    )(x, cmf)
    BR = 1000
    grid = N // BR
    row = lambda shp: pl.BlockSpec(shp, lambda i: (i, 0))
    rep = lambda shp: pl.BlockSpec(shp, lambda i: (0, 0))
    u1, u2, xwr = pl.pallas_call(
        _dense_body,
        grid=(grid,),
        in_specs=[
            row((BR, D)), row((BR, 1)), rep((1, D)), rep((1, D)),
            rep((D, D)), rep((D, D)), rep((1, D)), rep((D, D)), rep((1, D)),
            rep((2 * D, 1)), rep((2 * D, 1)),
        ],
        out_specs=[row((BR, D))] * 3,
        out_shape=[jax.ShapeDtypeStruct((N, D), jnp.float32)] * 3,
    )(x, cmf, mc, mn, W_r, W_s, b_s, W_t, b_t, Wg_s2t, Wg_t2s)
    return u1, u2, xwr


def _attn_body(u1, s1, d1, wf1, u2, s2, d2, wf2, w1o, w2o,
               idx_s, idx_d, rows_s, rows_d, wfv, wbuf, sem1, sem2):
    c = lax.axis_index("c")
    s = lax.axis_index("s")
    wid = s * 2 + c
    lane = lax.iota(jnp.int32, 16)

    for (u, se, de, wf, wo) in ((u1, s1, d1, wf1, w1o),
                                (u2, s2, d2, wf2, w2o)):
        pltpu.sync_copy(wf, wfv)

        def chunk(ci, _):
            base = wid * EPW + ci * KA
            pltpu.sync_copy(se.at[pl.ds(base, KA)], idx_s)
            pltpu.sync_copy(de.at[pl.ds(base, KA)], idx_d)
            for g in range(KA // 16):
                idx_d[pl.ds(g * 16, 16)] = jnp.minimum(
                    idx_d[pl.ds(g * 16, 16)], N - 1)
            cp1 = pltpu.async_copy(u.at[idx_s], rows_s, sem1)
            cp2 = pltpu.async_copy(u.at[idx_d], rows_d, sem2)
            cp1.wait()
            cp2.wait()

            def group(g, _):
                lv = jnp.zeros((16,), jnp.float32)
                for e in range(16):
                    a = jnp.zeros((16,), jnp.float32)
                    for j in range(D // 16):
                        v = (rows_s[g * 16 + e, pl.ds(j * 16, 16)]
                             + rows_d[g * 16 + e, pl.ds(j * 16, 16)])
                        a = a + jnp.maximum(v, 0.1 * v) * wfv[pl.ds(j * 16, 16)]
                    lv = jnp.where(lane == e, jnp.sum(a), lv)
                wbuf[pl.ds(g * 16, 16)] = jnp.exp(lv)
                return 0
            lax.fori_loop(0, KA // 16, group, 0)
            pltpu.sync_copy(wbuf, wo.at[pl.ds(base, KA)])
            return 0
        lax.fori_loop(0, EPW // KA, chunk, 0)


@functools.cache
def _make_attn_call():
    return functools.partial(
        pl.kernel,
        out_type=[jax.ShapeDtypeStruct((EP,), jnp.float32)] * 2,
        mesh=plsc.VectorSubcoreMesh(core_axis_name="c", subcore_axis_name="s",
                                    num_cores=2, num_subcores=16),
        compiler_params=pltpu.CompilerParams(needs_layout_passes=False),
        scratch_types=[
            pltpu.VMEM((KA,), jnp.int32),
            pltpu.VMEM((KA,), jnp.int32),
            pltpu.VMEM((KA, D), jnp.float32),
            pltpu.VMEM((KA, D), jnp.float32),
            pltpu.VMEM((D,), jnp.float32),
            pltpu.VMEM((KA,), jnp.float32),
            pltpu.SemaphoreType.DMA,
            pltpu.SemaphoreType.DMA,
        ],
    )(_attn_body)


def _agg_body(s1, d1, w1, s2, d2, w2, u1, u2, xwr, out,
              scan_s, scan_d, scan_w, pend_s, pend_l, pend_w, gidx,
              grows, acc, den, xbuf, sem1, sem2):
    c = lax.axis_index("c")
    s = lax.axis_index("s")
    wid = s * 2 + c
    lo = wid * SR
    lane = lax.iota(jnp.int32, 16)
    zz = jnp.zeros((16,), jnp.float32)

    def zr(r, _):
        for j in range(D // 16):
            acc[r, pl.ds(j * 16, 16)] = zz
        return 0
    lax.fori_loop(0, SR, zr, 0)
    for g in range(SR // 16):
        den[pl.ds(g * 16, 16)] = zz
    zi = jnp.zeros((16,), jnp.int32)
    for g in range(5):
        pend_s[pl.ds(g * 16, 16)] = zi
        pend_l[pl.ds(g * 16, 16)] = zi
        pend_w[pl.ds(g * 16, 16)] = zz

    def flush(u, wmask_fill):
        for g2 in range(4):
            gidx[pl.ds(g2 * 16, 16)] = pend_s[pl.ds(g2 * 16, 16)]
        pltpu.async_copy(u.at[gidx], grows, sem1).wait()
        for g2 in range(4):
            dl16 = pend_l[pl.ds(g2 * 16, 16)]
            w16 = pend_w[pl.ds(g2 * 16, 16)]
            if wmask_fill is not None:
                w16 = jnp.where(g2 * 16 + lane < wmask_fill, w16, 0.0)
            plsc.addupdate_scatter(den, [dl16], w16)
            e16 = g2 * 16 + lane

            def fbody(fo, _):
                for o in range(8):
                    fv = jnp.broadcast_to(fo * 8 + o, (16,))
                    vals = plsc.load_gather(grows, [e16, fv]) * w16
                    plsc.addupdate_scatter(acc, [dl16, fv], vals)
                return 0
            lax.fori_loop(0, D // 8, fbody, 0)

    for (se, de, we, u) in ((s1, d1, w1, u1), (s2, d2, w2, u2)):
        def scan_chunk(ci, fill):
            base = ci * SCN
            pltpu.sync_copy(se.at[pl.ds(base, SCN)], scan_s)
            pltpu.sync_copy(de.at[pl.ds(base, SCN)], scan_d)
            pltpu.sync_copy(we.at[pl.ds(base, SCN)], scan_w)

            def group(gi, fill):
                d16 = scan_d[pl.ds(gi * 16, 16)]
                dl = d16 - lo
                m = (dl >= 0) & (dl < SR)
                plsc.store_compressed(pend_s.at[pl.ds(fill, 16)],
                                      scan_s[pl.ds(gi * 16, 16)], mask=m)
                plsc.store_compressed(pend_l.at[pl.ds(fill, 16)], dl, mask=m)
                plsc.store_compressed(pend_w.at[pl.ds(fill, 16)],
                                      scan_w[pl.ds(gi * 16, 16)], mask=m)
                cnt = plsc.all_reduce_population_count(m)[0]
                nfill = fill + cnt

                @pl.when(nfill >= 64)
                def _():
                    flush(u, None)
                    for pref in (pend_s, pend_l, pend_w):
                        t = pref[pl.ds(64, 16)]
                        pref[pl.ds(0, 16)] = t

                return jnp.where(nfill >= 64, nfill - 64, nfill)
            return lax.fori_loop(0, SCN // 16, group, fill)
        fill = lax.fori_loop(0, EP // SCN, scan_chunk, 0)

        @pl.when(fill > 0)
        def _():
            flush(u, fill)

    for b in range(SR // 64):
        gbase = lo + b * 64

        def fin_rows(nrows_g, b=b):
            def fing(g, _):
                dv = den[pl.ds(b * 64 + g * 16, 16)]
                inv16 = 1.0 / (dv + 1e-16)
                for ln in range(16):
                    iv = jnp.broadcast_to(inv16[ln], (16,))

                    def jb(j, _):
                        xbuf[g * 16 + ln, pl.ds(j * 16, 16)] = (
                            acc[b * 64 + g * 16 + ln, pl.ds(j * 16, 16)] * iv
                            + xbuf[g * 16 + ln, pl.ds(j * 16, 16)])
                        return 0
                    lax.fori_loop(0, D // 16, jb, 0)
                return 0
            lax.fori_loop(0, nrows_g, fing, 0)

        @pl.when(gbase + 64 <= N)
        def _(b=b, gbase=gbase, fin_rows=fin_rows):
            pltpu.sync_copy(xwr.at[pl.ds(gbase, 64)], xbuf)
            fin_rows(4)
            pltpu.sync_copy(xbuf, out.at[pl.ds(gbase, 64)])

        @pl.when((gbase < N) & (gbase + 64 > N))
        def _(b=b, gbase=gbase, fin_rows=fin_rows):
            pltpu.sync_copy(xwr.at[pl.ds(gbase, 16)], xbuf.at[pl.ds(0, 16)])
            fin_rows(1)
            pltpu.sync_copy(xbuf.at[pl.ds(0, 16)], out.at[pl.ds(gbase, 16)])


@functools.cache
def _make_agg_call():
    return functools.partial(
        pl.kernel,
        out_type=jax.ShapeDtypeStruct((N, D), jnp.float32),
        mesh=plsc.VectorSubcoreMesh(core_axis_name="c", subcore_axis_name="s",
                                    num_cores=2, num_subcores=16),
        compiler_params=pltpu.CompilerParams(needs_layout_passes=False),
        scratch_types=[
            pltpu.VMEM((SCN,), jnp.int32),
            pltpu.VMEM((SCN,), jnp.int32),
            pltpu.VMEM((SCN,), jnp.float32),
            pltpu.VMEM((80,), jnp.int32),
            pltpu.VMEM((80,), jnp.int32),
            pltpu.VMEM((80,), jnp.float32),
            pltpu.VMEM((64,), jnp.int32),
            pltpu.VMEM((64, D), jnp.float32),
            pltpu.VMEM((SR, D), jnp.float32),
            pltpu.VMEM((SR,), jnp.float32),
            pltpu.VMEM((64, D), jnp.float32),
            pltpu.SemaphoreType.DMA,
            pltpu.SemaphoreType.DMA,
        ],
    )(_agg_body)


def _pad_edges(ei):
    src = jnp.concatenate([ei[0], jnp.zeros((EP - E,), jnp.int32)])
    dst = jnp.concatenate([ei[1], jnp.full((EP - E,), PAD_DST, jnp.int32)])
    return src, dst


def kernel(x, edge_index, edge_index1, edge_index2, central_mask,
           W_r, W_s, b_s, W_t, b_t, Wg_s2t, Wg_t2s, Wf_s2t, Wf_t2s):
    cmf = central_mask.astype(jnp.float32)[:, None]
    u1, u2, xwr = _dense_prep(x, cmf, W_r, W_s, b_s.reshape(1, D),
                              W_t, b_t.reshape(1, D), Wg_s2t, Wg_t2s)
    s1, d1 = _pad_edges(edge_index1)
    s2, d2 = _pad_edges(edge_index2)
    w1, w2 = _make_attn_call()(u1, s1, d1, Wf_t2s.reshape(D),
                               u2, s2, d2, Wf_s2t.reshape(D))
    return _make_agg_call()(s1, d1, w1, s2, d2, w2, u1, u2, xwr)


# speedup vs baseline: 1.5041x; 1.3456x over previous
import functools

import jax
import jax.numpy as jnp
from jax import lax
from jax.experimental import pallas as pl
from jax.experimental.pallas import tpu as pltpu
from jax.experimental.pallas import tpu_sc as plsc

N = 10000
D = 256
E = 80000
EP = 81920
NW = 32
EPW = EP // NW
KA = 64
SCN = 2048
SR = 320
PAD_DST = 16000


def _stats_body(x_ref, cm_ref, mc_ref, mn_ref):
    x = x_ref[...]
    cm = cm_ref[...]
    sc = jnp.sum(cm)
    mc_ref[...] = (jnp.sum(x * cm, axis=0) / jnp.maximum(sc, 1.0))[None, :]
    mn_ref[...] = (jnp.sum(x * (1.0 - cm), axis=0)
                   / jnp.maximum(float(N) - sc, 1.0))[None, :]


def _dense_body(x_ref, cm_ref, mc_ref, mn_ref, wr_ref, ws_ref, bs_ref,
                wt_ref, bt_ref, wg1_ref, wg2_ref, u1_ref, u2_ref, xwr_ref):
    x = x_ref[...]
    cm = cm_ref[...]
    delta = mc_ref[0, :] - mn_ref[0, :]
    wg1 = wg1_ref[...]
    wg2 = wg2_ref[...]
    c1 = jnp.sum(delta * wg1[D:, 0])
    c2 = jnp.sum(delta * wg2[D:, 0])
    g1 = jnp.dot(x, wg1[:D])
    g2 = jnp.dot(x, wg2[:D])
    sh1 = jnp.tanh(g1 + c1) * delta[None, :]
    sh2 = jnp.tanh(g2 + c2) * delta[None, :]
    xs = x - sh1 * cm
    xt = x + sh2 * (1.0 - cm)
    u1_ref[...] = jnp.dot(xt, ws_ref[...]) + bs_ref[...]
    u2_ref[...] = jnp.dot(xs, wt_ref[...]) + bt_ref[...]
    xwr_ref[...] = jnp.dot(x, wr_ref[...])


def _dense_prep(x, cmf, W_r, W_s, b_s, W_t, b_t, Wg_s2t, Wg_t2s):
    mc, mn = pl.pallas_call(
        _stats_body,
        out_shape=[jax.ShapeDtypeStruct((1, D), jnp.float32)] * 2,
    )(x, cmf)
    BR = 1000
    grid = N // BR
    row = lambda shp: pl.BlockSpec(shp, lambda i: (i, 0))
    rep = lambda shp: pl.BlockSpec(shp, lambda i: (0, 0))
    u1, u2, xwr = pl.pallas_call(
        _dense_body,
        grid=(grid,),
        in_specs=[
            row((BR, D)), row((BR, 1)), rep((1, D)), rep((1, D)),
            rep((D, D)), rep((D, D)), rep((1, D)), rep((D, D)), rep((1, D)),
            rep((2 * D, 1)), rep((2 * D, 1)),
        ],
        out_specs=[row((BR, D))] * 3,
        out_shape=[jax.ShapeDtypeStruct((N, D), jnp.float32)] * 3,
    )(x, cmf, mc, mn, W_r, W_s, b_s, W_t, b_t, Wg_s2t, Wg_t2s)
    return u1, u2, xwr


def _attn_body(u1, s1, d1, wf1, u2, s2, d2, wf2, w1o, w2o,
               idxs, idxd, ib0, ib1, rb0, rb1, wfv, wfull, sg0, sg1):
    c = lax.axis_index("c")
    s = lax.axis_index("s")
    wid = s * 2 + c
    lane = lax.iota(jnp.int32, 16)
    NCH = EPW // KA
    islots = (ib0, ib1)
    rslots = (rb0, rb1)
    gsems = (sg0, sg1)

    for (u, se, de, wf, wo) in ((u1, s1, d1, wf1, w1o),
                                (u2, s2, d2, wf2, w2o)):
        pltpu.sync_copy(wf, wfv)
        pltpu.sync_copy(se.at[pl.ds(wid * EPW, EPW)], idxs)
        pltpu.sync_copy(de.at[pl.ds(wid * EPW, EPW)], idxd)

        def stage(ci, b):
            off = ci * KA
            for g in range(KA // 16):
                islots[b][pl.ds(g * 16, 16)] = idxs[pl.ds(off + g * 16, 16)]
                islots[b][pl.ds(KA + g * 16, 16)] = jnp.minimum(
                    idxd[pl.ds(off + g * 16, 16)], N - 1)
            pltpu.make_async_copy(u.at[islots[b]], rslots[b], gsems[b]).start()

        for b in range(2):
            stage(b, b)

        def pair(p, _):
            for b in range(2):
                ci = p * 2 + b
                pltpu.make_async_copy(u.at[islots[b]], rslots[b],
                                      gsems[b]).wait()
                rows = rslots[b]

                def group(g, _):
                    def edge(e, lv):
                        a = jnp.zeros((16,), jnp.float32)
                        for j in range(D // 16):
                            v = (rows[g * 16 + e, pl.ds(j * 16, 16)]
                                 + rows[KA + g * 16 + e, pl.ds(j * 16, 16)])
                            a = (a + jnp.maximum(v, 0.1 * v)
                                 * wfv[pl.ds(j * 16, 16)])
                        return jnp.where(lane == e, jnp.sum(a), lv)
                    lv = lax.fori_loop(0, 16, edge,
                                       jnp.zeros((16,), jnp.float32))
                    wfull[pl.ds(ci * KA + g * 16, 16)] = jnp.exp(lv)
                    return 0
                lax.fori_loop(0, KA // 16, group, 0)

                @pl.when(ci + 2 < NCH)
                def _(b=b, ci=ci):
                    stage(ci + 2, b)
            return 0
        lax.fori_loop(0, NCH // 2, pair, 0)
        pltpu.sync_copy(wfull, wo.at[pl.ds(wid * EPW, EPW)])


@functools.cache
def _make_attn_call():
    return functools.partial(
        pl.kernel,
        out_type=[jax.ShapeDtypeStruct((EP,), jnp.float32)] * 2,
        mesh=plsc.VectorSubcoreMesh(core_axis_name="c", subcore_axis_name="s",
                                    num_cores=2, num_subcores=16),
        compiler_params=pltpu.CompilerParams(needs_layout_passes=False),
        scratch_types=[
            pltpu.VMEM((EPW,), jnp.int32),
            pltpu.VMEM((EPW,), jnp.int32),
            pltpu.VMEM((2 * KA,), jnp.int32),
            pltpu.VMEM((2 * KA,), jnp.int32),
            pltpu.VMEM((2 * KA, D), jnp.float32),
            pltpu.VMEM((2 * KA, D), jnp.float32),
            pltpu.VMEM((D,), jnp.float32),
            pltpu.VMEM((EPW,), jnp.float32),
            pltpu.SemaphoreType.DMA,
            pltpu.SemaphoreType.DMA,
        ],
    )(_attn_body)


def _agg_body(pa, wa, ua, xwr, out,
              sp0, sp1, sw0, sw1, pend_p, pend_w, gidx,
              stg_l, stg_w, grows, acc, den, xbuf, sem1, sc0, sc1):
    c = lax.axis_index("c")
    s = lax.axis_index("s")
    wid = s * 2 + c
    lo = wid * SR
    lane = lax.iota(jnp.int32, 16)
    zz = jnp.zeros((16,), jnp.float32)

    def zr(r, _):
        for j in range(D // 16):
            acc[r, pl.ds(j * 16, 16)] = zz
        return 0
    lax.fori_loop(0, SR, zr, 0)
    for g in range(SR // 16):
        den[pl.ds(g * 16, 16)] = zz
    zi = jnp.zeros((16,), jnp.int32)
    for g in range(9):
        pend_p[pl.ds(g * 16, 16)] = zi
        pend_w[pl.ds(g * 16, 16)] = zz

    def stage_flush(wmask_fill):
        for g2 in range(4):
            pk16 = pend_p[pl.ds(g2 * 16, 16)]
            gidx[pl.ds(g2 * 16, 16)] = pk16 & 0x7FFF
            stg_l[pl.ds(g2 * 16, 16)] = lax.shift_right_logical(pk16, 15)
            w16 = pend_w[pl.ds(g2 * 16, 16)]
            if wmask_fill is not None:
                w16 = jnp.where(g2 * 16 + lane < wmask_fill, w16, 0.0)
            stg_w[pl.ds(g2 * 16, 16)] = w16
        pltpu.make_async_copy(ua.at[gidx], grows, sem1).start()

    def complete_flush():
        pltpu.make_async_copy(ua.at[gidx], grows, sem1).wait()
        for g2 in range(4):
            dl16 = stg_l[pl.ds(g2 * 16, 16)]
            w16 = stg_w[pl.ds(g2 * 16, 16)]
            plsc.addupdate_scatter(den, [dl16], w16)
            e16 = g2 * 16 + lane

            def fbody(fo, _):
                for o in range(8):
                    fv = jnp.broadcast_to(fo * 8 + o, (16,))
                    vals = plsc.load_gather(grows, [e16, fv]) * w16
                    plsc.addupdate_scatter(acc, [dl16, fv], vals)
                return 0
            lax.fori_loop(0, D // 8, fbody, 0)

    pslots = (sp0, sp1)
    wslots = (sw0, sw1)
    csems = (sc0, sc1)
    NCH = 2 * EP // SCN

    def stage(ci, b):
        base = ci * SCN
        pltpu.make_async_copy(pa.at[pl.ds(base, SCN)], pslots[b],
                              csems[b]).start()
        pltpu.make_async_copy(wa.at[pl.ds(base, SCN)], wslots[b],
                              csems[b]).start()

    for b in range(2):
        stage(b, b)

    def pair(p, st):
        for b in range(2):
            ci = p * 2 + b
            pltpu.make_async_copy(pa.at[pl.ds(0, SCN)], pslots[b],
                                  csems[b]).wait()
            pltpu.make_async_copy(wa.at[pl.ds(0, SCN)], wslots[b],
                                  csems[b]).wait()
            scan_p = pslots[b]
            scan_w = wslots[b]

            def quad(qi, st):
                fill, outst = st
                for k in range(4):
                    gi = qi * 4 + k
                    pk = scan_p[pl.ds(gi * 16, 16)] - (lo << 15)
                    m = (pk >= 0) & (pk < (SR << 15))
                    plsc.store_compressed(pend_p.at[pl.ds(fill, 16)], pk,
                                          mask=m)
                    plsc.store_compressed(pend_w.at[pl.ds(fill, 16)],
                                          scan_w[pl.ds(gi * 16, 16)],
                                          mask=m)
                    fill = fill + plsc.all_reduce_population_count(m)[0]
                trig = fill >= 64

                @pl.when(trig)
                def _():
                    @pl.when(outst > 0)
                    def _():
                        complete_flush()
                    stage_flush(None)
                    for pref in (pend_p, pend_w):
                        for g in range(4):
                            t = pref[pl.ds(64 + g * 16, 16)]
                            pref[pl.ds(g * 16, 16)] = t

                return (jnp.where(trig, fill - 64, fill),
                        jnp.where(trig, jnp.int32(1), outst))
            st = lax.fori_loop(0, SCN // 64, quad, st)

            @pl.when(ci + 2 < NCH)
            def _(b=b, ci=ci):
                stage(ci + 2, b)
        return st
    fill, outst = lax.fori_loop(0, NCH // 2, pair,
                                (jnp.int32(0), jnp.int32(0)))

    @pl.when(outst > 0)
    def _():
        complete_flush()

    @pl.when(fill > 0)
    def _():
        stage_flush(fill)
        complete_flush()

    for b in range(SR // 64):
        gbase = lo + b * 64

        def fin_rows(nrows_g, b=b):
            def fing(g, _):
                dv = den[pl.ds(b * 64 + g * 16, 16)]
                inv16 = 1.0 / (dv + 1e-16)
                for ln in range(16):
                    iv = jnp.broadcast_to(inv16[ln], (16,))

                    def jb(j, _):
                        xbuf[g * 16 + ln, pl.ds(j * 16, 16)] = (
                            acc[b * 64 + g * 16 + ln, pl.ds(j * 16, 16)] * iv
                            + xbuf[g * 16 + ln, pl.ds(j * 16, 16)])
                        return 0
                    lax.fori_loop(0, D // 16, jb, 0)
                return 0
            lax.fori_loop(0, nrows_g, fing, 0)

        @pl.when(gbase + 64 <= N)
        def _(b=b, gbase=gbase, fin_rows=fin_rows):
            pltpu.sync_copy(xwr.at[pl.ds(gbase, 64)], xbuf)
            fin_rows(4)
            pltpu.sync_copy(xbuf, out.at[pl.ds(gbase, 64)])

        @pl.when((gbase < N) & (gbase + 64 > N))
        def _(b=b, gbase=gbase, fin_rows=fin_rows):
            pltpu.sync_copy(xwr.at[pl.ds(gbase, 16)], xbuf.at[pl.ds(0, 16)])
            fin_rows(1)
            pltpu.sync_copy(xbuf.at[pl.ds(0, 16)], out.at[pl.ds(gbase, 16)])


@functools.cache
def _make_agg_call():
    return functools.partial(
        pl.kernel,
        out_type=jax.ShapeDtypeStruct((N, D), jnp.float32),
        mesh=plsc.VectorSubcoreMesh(core_axis_name="c", subcore_axis_name="s",
                                    num_cores=2, num_subcores=16),
        compiler_params=pltpu.CompilerParams(needs_layout_passes=False),
        scratch_types=[
            pltpu.VMEM((SCN,), jnp.int32),
            pltpu.VMEM((SCN,), jnp.int32),
            pltpu.VMEM((SCN,), jnp.float32),
            pltpu.VMEM((SCN,), jnp.float32),
            pltpu.VMEM((144,), jnp.int32),
            pltpu.VMEM((144,), jnp.float32),
            pltpu.VMEM((64,), jnp.int32),
            pltpu.VMEM((64,), jnp.int32),
            pltpu.VMEM((64,), jnp.float32),
            pltpu.VMEM((64, D), jnp.float32),
            pltpu.VMEM((SR, D), jnp.float32),
            pltpu.VMEM((SR,), jnp.float32),
            pltpu.VMEM((64, D), jnp.float32),
            pltpu.SemaphoreType.DMA,
            pltpu.SemaphoreType.DMA,
            pltpu.SemaphoreType.DMA,
        ],
    )(_agg_body)


def _pad_edges(ei):
    src = jnp.concatenate([ei[0], jnp.zeros((EP - E,), jnp.int32)])
    dst = jnp.concatenate([ei[1], jnp.full((EP - E,), PAD_DST, jnp.int32)])
    return src, dst


def kernel(x, edge_index, edge_index1, edge_index2, central_mask,
           W_r, W_s, b_s, W_t, b_t, Wg_s2t, Wg_t2s, Wf_s2t, Wf_t2s):
    cmf = central_mask.astype(jnp.float32)[:, None]
    u1, u2, xwr = _dense_prep(x, cmf, W_r, W_s, b_s.reshape(1, D),
                              W_t, b_t.reshape(1, D), Wg_s2t, Wg_t2s)
    s1, d1 = _pad_edges(edge_index1)
    s2, d2 = _pad_edges(edge_index2)
    w1, w2 = _make_attn_call()(u1, s1, d1, Wf_t2s.reshape(D),
                               u2, s2, d2, Wf_s2t.reshape(D))
    pa = jnp.concatenate([s1 | (d1 << 15), (s2 + N) | (d2 << 15)])
    wa = jnp.concatenate([w1, w2])
    ua = jnp.concatenate([u1, u2], axis=0)
    return _make_agg_call()(pa, wa, ua, xwr)
